# Initial kernel scaffold; baseline (speedup 1.0000x reference)
import jax
import jax.numpy as jnp
from jax import lax
from jax.experimental import pallas as pl
from jax.experimental.pallas import tpu as pltpu
from jax.experimental.pallas import tpu_sc as plsc

B = 16384
D = 64
W = 20
NW = 32
BPW = B // NW
C = 32
NCHUNK = BPW // C
IDX_PER_CHUNK = C * W
SUB = 128
NSUB = IDX_PER_CHUNK // SUB


def _body(walk_hbm, neg_hbm, start_hbm, emb_hbm, outp_hbm, outn_hbm,
          widx, nidx, sidx, wrows, nrows, srows, st_t, outp_v, outn_v, sem):
    nc = 2
    wid = lax.axis_index("s") * nc + lax.axis_index("c")
    lanes = lax.iota(jnp.int32, 16)

    def chunk_body(c, _):
        base = wid * BPW + c * C
        ibase = base * W

        pltpu.sync_copy(walk_hbm.at[pl.ds(ibase, IDX_PER_CHUNK)], widx)
        pltpu.sync_copy(neg_hbm.at[pl.ds(ibase, IDX_PER_CHUNK)], nidx)
        pltpu.sync_copy(start_hbm.at[pl.ds(base, C)], sidx)

        copies = []
        for r in range(NSUB):
            sl = pl.ds(r * SUB, SUB)
            copies.append(pltpu.make_async_copy(
                emb_hbm.at[widx.at[sl]], wrows.at[sl], sem))
            copies.append(pltpu.make_async_copy(
                emb_hbm.at[nidx.at[sl]], nrows.at[sl], sem))
        copies.append(pltpu.make_async_copy(emb_hbm.at[sidx], srows, sem))
        for cp in copies:
            cp.start()
        for cp in copies:
            cp.wait()

        for g in range(2):
            row_idx = g * 16 + lanes
            for d in range(D):
                col = jnp.full((16,), d, jnp.int32)
                v = plsc.load_gather(srows, [row_idx, col])
                st_t[d, pl.ds(g * 16, 16)] = v

        def j_body(j, _):
            for g in range(2):
                brow = (g * 16 + lanes) * W + j
                accw = jnp.zeros((16,), jnp.float32)
                accn = jnp.zeros((16,), jnp.float32)
                for d in range(D):
                    col = jnp.full((16,), d, jnp.int32)
                    sv = st_t[d, pl.ds(g * 16, 16)]
                    wv = plsc.load_gather(wrows, [brow, col])
                    nv = plsc.load_gather(nrows, [brow, col])
                    accw = accw + sv * wv
                    accn = accn + sv * nv
                pw = 1.0 / (1.0 + jnp.exp(-accw))
                pn = 1.0 / (1.0 + jnp.exp(accn))
                plsc.store_scatter(outp_v, [brow], pw)
                plsc.store_scatter(outn_v, [brow], pn)
            return 0

        lax.fori_loop(0, W, j_body, 0)

        pltpu.sync_copy(outp_v, outp_hbm.at[pl.ds(ibase, IDX_PER_CHUNK)])
        pltpu.sync_copy(outn_v, outn_hbm.at[pl.ds(ibase, IDX_PER_CHUNK)])
        return 0

    lax.fori_loop(0, NCHUNK, chunk_body, 0)


@jax.jit
def _run(walk_flat, neg_flat, start_node, embeddings):
    mesh = plsc.VectorSubcoreMesh(core_axis_name="c", subcore_axis_name="s")
    f = pl.kernel(
        _body,
        mesh=mesh,
        out_type=(
            jax.ShapeDtypeStruct((B * W,), jnp.float32),
            jax.ShapeDtypeStruct((B * W,), jnp.float32),
        ),
        scratch_types=[
            pltpu.VMEM((IDX_PER_CHUNK,), jnp.int32),
            pltpu.VMEM((IDX_PER_CHUNK,), jnp.int32),
            pltpu.VMEM((C,), jnp.int32),
            pltpu.VMEM((IDX_PER_CHUNK, D), jnp.float32),
            pltpu.VMEM((IDX_PER_CHUNK, D), jnp.float32),
            pltpu.VMEM((C, D), jnp.float32),
            pltpu.VMEM((D, C), jnp.float32),
            pltpu.VMEM((IDX_PER_CHUNK,), jnp.float32),
            pltpu.VMEM((IDX_PER_CHUNK,), jnp.float32),
            pltpu.SemaphoreType.DMA,
        ],
    )
    return f(walk_flat, neg_flat, start_node, embeddings)


def kernel(start_node, walk, negative_samples, embeddings):
    walk_flat = walk.reshape(-1)
    neg_flat = negative_samples.reshape(-1)
    pos_flat, negp_flat = _run(walk_flat, neg_flat, start_node, embeddings)
    return (pos_flat.reshape(B, W), negp_flat.reshape(B, W))


# speedup vs baseline: 1.2157x; 1.2157x over previous
import jax
import jax.numpy as jnp
from jax import lax
from jax.experimental import pallas as pl
from jax.experimental.pallas import tpu as pltpu
from jax.experimental.pallas import tpu_sc as plsc

B = 16384
D = 64
W = 20
NW = 32
BPW = B // NW
C = 32
NCHUNK = BPW // C
IDX_PER_CHUNK = C * W
SUB = 128
NSUB = IDX_PER_CHUNK // SUB
NVEC = D // 16


def _body(walk_hbm, neg_hbm, start_hbm, emb_hbm, outp_hbm, outn_hbm,
          widx, nidx, sidx, wrows, nrows, srows, outp_v, outn_v, sem):
    nc = 2
    wid = lax.axis_index("s") * nc + lax.axis_index("c")

    def chunk_body(c, _):
        base = wid * BPW + c * C
        ibase = base * W

        pltpu.sync_copy(walk_hbm.at[pl.ds(ibase, IDX_PER_CHUNK)], widx)
        pltpu.sync_copy(neg_hbm.at[pl.ds(ibase, IDX_PER_CHUNK)], nidx)
        pltpu.sync_copy(start_hbm.at[pl.ds(base, C)], sidx)

        copies = []
        for r in range(NSUB):
            sl = pl.ds(r * SUB, SUB)
            copies.append(pltpu.make_async_copy(
                emb_hbm.at[widx.at[sl]], wrows.at[sl], sem))
            copies.append(pltpu.make_async_copy(
                emb_hbm.at[nidx.at[sl]], nrows.at[sl], sem))
        copies.append(pltpu.make_async_copy(emb_hbm.at[sidx], srows, sem))
        for cp in copies:
            cp.start()
        for cp in copies:
            cp.wait()

        lanes = lax.iota(jnp.int32, 16)
        last_lane = lanes == 15

        def b_body(b, _):
            sv = [srows[b, pl.ds(k * 16, 16)] for k in range(NVEC)]
            rbase = b * W
            for j in range(W):
                r = rbase + j
                accw = sv[0] * wrows[r, pl.ds(0, 16)]
                accn = sv[0] * nrows[r, pl.ds(0, 16)]
                for k in range(1, NVEC):
                    accw = accw + sv[k] * wrows[r, pl.ds(k * 16, 16)]
                    accn = accn + sv[k] * nrows[r, pl.ds(k * 16, 16)]
                plsc.store_compressed(outp_v.at[pl.ds(r, 16)],
                                      plsc.cumsum(accw), mask=last_lane)
                plsc.store_compressed(outn_v.at[pl.ds(r, 16)],
                                      plsc.cumsum(accn), mask=last_lane)
            return 0

        lax.fori_loop(0, C, b_body, 0)

        for k in range(IDX_PER_CHUNK // 16):
            sl = pl.ds(k * 16, 16)
            outp_v[sl] = 1.0 / (1.0 + jnp.exp(-outp_v[sl]))
            outn_v[sl] = 1.0 / (1.0 + jnp.exp(outn_v[sl]))

        pltpu.sync_copy(outp_v.at[pl.ds(0, IDX_PER_CHUNK)],
                        outp_hbm.at[pl.ds(ibase, IDX_PER_CHUNK)])
        pltpu.sync_copy(outn_v.at[pl.ds(0, IDX_PER_CHUNK)],
                        outn_hbm.at[pl.ds(ibase, IDX_PER_CHUNK)])
        return 0

    lax.fori_loop(0, NCHUNK, chunk_body, 0)


@jax.jit
def _run(walk_flat, neg_flat, start_node, embeddings):
    mesh = plsc.VectorSubcoreMesh(core_axis_name="c", subcore_axis_name="s")
    f = pl.kernel(
        _body,
        mesh=mesh,
        compiler_params=pltpu.CompilerParams(
            needs_layout_passes=False, use_tc_tiling_on_sc=False),
        out_type=(
            jax.ShapeDtypeStruct((B * W,), jnp.float32),
            jax.ShapeDtypeStruct((B * W,), jnp.float32),
        ),
        scratch_types=[
            pltpu.VMEM((IDX_PER_CHUNK,), jnp.int32),
            pltpu.VMEM((IDX_PER_CHUNK,), jnp.int32),
            pltpu.VMEM((C,), jnp.int32),
            pltpu.VMEM((IDX_PER_CHUNK, D), jnp.float32),
            pltpu.VMEM((IDX_PER_CHUNK, D), jnp.float32),
            pltpu.VMEM((C, D), jnp.float32),
            pltpu.VMEM((IDX_PER_CHUNK + 16,), jnp.float32),
            pltpu.VMEM((IDX_PER_CHUNK + 16,), jnp.float32),
            pltpu.SemaphoreType.DMA,
        ],
    )
    return f(walk_flat, neg_flat, start_node, embeddings)


def kernel(start_node, walk, negative_samples, embeddings):
    walk_flat = walk.reshape(-1)
    neg_flat = negative_samples.reshape(-1)
    pos_flat, negp_flat = _run(walk_flat, neg_flat, start_node, embeddings)
    return (pos_flat.reshape(B, W), negp_flat.reshape(B, W))
